# Initial kernel scaffold; baseline (speedup 1.0000x reference)
import jax
import jax.numpy as jnp
from jax.experimental import pallas as pl


def kernel(slot_vals, slot_lengths, embed_table, W, b):
    raise NotImplementedError("write your pallas kernel here")


# speedup vs baseline: 9.9392x; 9.9392x over previous
import functools

import jax
import jax.numpy as jnp
from jax import lax
from jax.experimental import pallas as pl
from jax.experimental.pallas import tpu as pltpu
from jax.experimental.pallas import tpu_sc as plsc

VOCAB = 100000
H = 128
B, S, L = 1024, 26, 20
N = B * S
NW = 32
SLOTS_PER_W = N // NW
CS = 16
GI = 80
NG = CS * L // GI
NCHUNK = SLOTS_PER_W // CS
LANES = 16
HV = H // LANES


def _sc_bow_sums(idx_flat, lens, table):
    mesh = plsc.VectorSubcoreMesh(core_axis_name="c", subcore_axis_name="s")

    @functools.partial(
        pl.kernel,
        mesh=mesh,
        out_type=jax.ShapeDtypeStruct((N, H), jnp.float32),
        scratch_types=[
            pltpu.VMEM((SLOTS_PER_W,), jnp.int32),
            pltpu.VMEM((SLOTS_PER_W * L,), jnp.int32),
            pltpu.VMEM((2, CS * L, H), jnp.float32),
            pltpu.VMEM((CS, H), jnp.float32),
            pltpu.SemaphoreType.DMA,
            pltpu.SemaphoreType.DMA,
        ],
    )
    def k(idx_hbm, lens_hbm, table_hbm, out_hbm,
          lens_v, idx_v, rows_v, out_v, sem0, sem1):
        wid = lax.axis_index("s") * 2 + lax.axis_index("c")
        slot0 = wid * SLOTS_PER_W
        pltpu.sync_copy(lens_hbm.at[pl.ds(slot0, SLOTS_PER_W)], lens_v)
        pltpu.sync_copy(idx_hbm.at[pl.ds(slot0 * L, SLOTS_PER_W * L)], idx_v)

        def fire(ci, buf, sem):
            i0 = ci * CS * L
            for g in range(NG):
                pltpu.async_copy(
                    table_hbm.at[idx_v.at[pl.ds(i0 + g * GI, GI)]],
                    rows_v.at[buf].at[pl.ds(g * GI, GI)], sem)

        def drain(buf, sem):
            for g in range(NG):
                pltpu.make_async_copy(
                    table_hbm.at[idx_v.at[pl.ds(g * GI, GI)]],
                    rows_v.at[buf].at[pl.ds(g * GI, GI)], sem).wait()

        def compute(ci, buf):
            lens16 = lens_v[pl.ds(ci * CS, LANES)]
            for jj in range(CS):
                len_s = lens16[jj]

                def l_body(l, accs):
                    return tuple(
                        accs[h] + rows_v[buf, jj * L + l, pl.ds(h * LANES, LANES)]
                        for h in range(HV))

                accs = lax.fori_loop(
                    0, len_s, l_body,
                    tuple(jnp.zeros((LANES,), jnp.float32) for _ in range(HV)))
                for h in range(HV):
                    out_v[jj, pl.ds(h * LANES, LANES)] = accs[h]
            pltpu.sync_copy(out_v, out_hbm.at[pl.ds(slot0 + ci * CS, CS)])

        fire(0, 0, sem0)

        def outer(co, carry):
            for d in range(2):
                ci = co * 2 + d
                sem_cur = sem0 if d == 0 else sem1
                sem_nxt = sem1 if d == 0 else sem0

                @pl.when(ci + 1 < NCHUNK)
                def _():
                    fire(ci + 1, 1 - d, sem_nxt)

                drain(d, sem_cur)
                compute(ci, d)
            return carry

        lax.fori_loop(0, NCHUNK // 2, outer, 0)

    return k(idx_flat, lens, table)


def _tc_linear(sums, cnt, W, b2):
    BLK = 1024

    def body(s_ref, c_ref, w_ref, b_ref, o_ref):
        s = s_ref[...]
        c = c_ref[...]
        mm = lax.dot_general(s, w_ref[...], (((1,), (1,)), ((), ())),
                             preferred_element_type=jnp.float32)
        o_ref[...] = (mm + c * b_ref[...]) * (1.0 / (c + 1e-5))

    return pl.pallas_call(
        body,
        grid=(N // BLK,),
        in_specs=[
            pl.BlockSpec((BLK, H), lambda i: (i, 0)),
            pl.BlockSpec((BLK, 1), lambda i: (i, 0)),
            pl.BlockSpec((H, H), lambda i: (0, 0)),
            pl.BlockSpec((1, H), lambda i: (0, 0)),
        ],
        out_specs=pl.BlockSpec((BLK, H), lambda i: (i, 0)),
        out_shape=jax.ShapeDtypeStruct((N, H), jnp.float32),
    )(sums, cnt, W, b2)


def kernel(slot_vals, slot_lengths, embed_table, W, b):
    idx_flat = slot_vals.reshape(-1)
    lens = slot_lengths.reshape(-1)
    sums = _sc_bow_sums(idx_flat, lens, embed_table)
    cnt = lens.astype(jnp.float32).reshape(N, 1)
    out = _tc_linear(sums, cnt, W, b.reshape(1, H))
    return out.reshape(B, S, H)
